# baseline (speedup 1.0000x reference)
import jax
import jax.numpy as jnp
from jax import lax
from jax.experimental import pallas as pl
from jax.experimental.pallas import tpu as pltpu

_K_TOP = 256
_CELLS_PER_ROW = 8


def _cellcnn_body(
    xw_ref, bd_ref, b1_ref, w2_ref, b2_ref, out_ref, hhi_ref, hlo_ref
):
    B, nr, nl = hhi_ref.shape
    nf = nl // _CELLS_PER_ROW
    step = pl.program_id(0)

    @pl.when(step < B)
    def _matmul():
        x = xw_ref[0]
        h = jnp.dot(x, bd_ref[...], preferred_element_type=jnp.float32)
        h = jnp.maximum(h + b1_ref[...], 0.0)
        bits = lax.bitcast_convert_type(h, jnp.int32)
        hhi_ref[step] = (bits >> 16).astype(jnp.int16)
        hlo_ref[step] = ((bits & 0xFFFF) - 32768).astype(jnp.int16)

    @pl.when(step == B)
    def _search():
        li = lax.broadcasted_iota(jnp.int32, (nl, nl), 0)
        mi = lax.broadcasted_iota(jnp.int32, (nl, nl), 1)
        foldm = jnp.where((li % nf) == (mi % nf), 1.0, 0.0).astype(jnp.float32)

        def fold(cnt_rows):
            cnt = jnp.concatenate(cnt_rows, axis=0).astype(jnp.float32)
            return jnp.dot(cnt, foldm, preferred_element_type=jnp.float32)

        kf = float(_K_TOP)

        def p1_body(_, carry):
            lo, hi = carry
            mid = lo + lax.div(hi - lo, 2)
            mid16 = mid.astype(jnp.int16)
            cnt = fold(
                [
                    jnp.sum(
                        (hhi_ref[b] >= mid16[b : b + 1]).astype(jnp.int16),
                        axis=0,
                        keepdims=True,
                    )
                    for b in range(B)
                ]
            )
            pred = cnt >= kf
            return jnp.where(pred, mid, lo), jnp.where(pred, hi, mid)

        lo1 = jnp.zeros((B, nl), jnp.int32)
        hi1 = jnp.full((B, nl), jnp.int32(32768))
        u, _ = lax.fori_loop(0, 15, p1_body, (lo1, hi1))
        u16 = u.astype(jnp.int16)

        c_hi = fold(
            [
                jnp.sum(
                    (hhi_ref[b] > u16[b : b + 1]).astype(jnp.int16),
                    axis=0,
                    keepdims=True,
                )
                for b in range(B)
            ]
        )
        k2 = kf - c_hi

        def p2_body(_, carry):
            lo, hi = carry
            mid = lo + lax.div(hi - lo, 2)
            w16 = (mid - 32768).astype(jnp.int16)
            cnt = fold(
                [
                    jnp.sum(
                        (
                            (hhi_ref[b] == u16[b : b + 1])
                            & (hlo_ref[b] >= w16[b : b + 1])
                        ).astype(jnp.int16),
                        axis=0,
                        keepdims=True,
                    )
                    for b in range(B)
                ]
            )
            pred = cnt >= k2
            return jnp.where(pred, mid, lo), jnp.where(pred, hi, mid)

        lo2 = jnp.zeros((B, nl), jnp.int32)
        hi2 = jnp.full((B, nl), jnp.int32(65536))
        w, _ = lax.fori_loop(0, 16, p2_body, (lo2, hi2))

        vk_bits = (u << 16) | w
        t_lo = lax.bitcast_convert_type(vk_bits, jnp.float32)

        CH2 = 512
        sums_l, cgt_l = [], []
        for b in range(B):
            s_acc = jnp.zeros((1, nl), jnp.float32)
            c_acc = jnp.zeros((1, nl), jnp.float32)
            for c in range(nr // CH2):
                sl = pl.ds(c * CH2, CH2)
                bits = (hhi_ref[b, sl, :].astype(jnp.int32) << 16) | (
                    (hlo_ref[b, sl, :].astype(jnp.int32) + 32768) & 0xFFFF
                )
                v = lax.bitcast_convert_type(bits, jnp.float32)
                mgt = bits > vk_bits[b : b + 1]
                s_acc = s_acc + jnp.sum(
                    jnp.where(mgt, v, 0.0), axis=0, keepdims=True
                )
                c_acc = c_acc + jnp.sum(
                    mgt.astype(jnp.float32), axis=0, keepdims=True
                )
            sums_l.append(s_acc)
            cgt_l.append(c_acc)
        sumsf = fold(sums_l)
        cgtf = fold(cgt_l)
        sum_top = sumsf + (kf - cgtf) * t_lo
        pooled = sum_top[:, :nf] * (1.0 / kf)

        z = jnp.sum(pooled * w2_ref[...], axis=1, keepdims=True) + b2_ref[...]
        out_ref[...] = (1.0 / (1.0 + jnp.exp(-z))).reshape(B, 1, 1)


def _build_call(B, NR, D, F):
    C = _CELLS_PER_ROW
    return pl.pallas_call(
        _cellcnn_body,
        grid=(B + 1,),
        in_specs=[
            pl.BlockSpec((1, NR, C * D), lambda b: (jnp.minimum(b, B - 1), 0, 0)),
            pl.BlockSpec((C * D, C * F), lambda b: (0, 0)),
            pl.BlockSpec((1, C * F), lambda b: (0, 0)),
            pl.BlockSpec((1, F), lambda b: (0, 0)),
            pl.BlockSpec((1, 1), lambda b: (0, 0)),
        ],
        out_specs=pl.BlockSpec((B, 1, 1), lambda b: (0, 0, 0)),
        out_shape=jax.ShapeDtypeStruct((B, 1, 1), jnp.float32),
        scratch_shapes=[
            pltpu.VMEM((B, NR, C * F), jnp.int16),
            pltpu.VMEM((B, NR, C * F), jnp.int16),
        ],
    )


def kernel(inputs, W1, b1, W2, b2):
    B, N, D = inputs.shape
    F = W1.shape[1]
    C = _CELLS_PER_ROW
    NR = N // C
    xw = inputs.reshape(B, NR, C * D)
    eye = jnp.eye(C, dtype=W1.dtype)
    bd = jnp.einsum("ce,df->cdef", eye, W1).reshape(C * D, C * F)
    b1t = jnp.tile(b1, C).reshape(1, C * F)
    w2t = W2.reshape(1, F)
    b2r = b2.reshape(1, 1)
    out = _build_call(B, NR, D, F)(xw, bd, b1t, w2t, b2r)
    return out.reshape(B, 1)


# speedup vs baseline: 1.2334x; 1.2334x over previous
import jax
import jax.numpy as jnp
from jax import lax
from jax.experimental import pallas as pl
from jax.experimental.pallas import tpu as pltpu

_K_TOP = 256
_CELLS_PER_ROW = 8
_PAIR = 2


def _cellcnn_body(xw_ref, bd_ref, b1_ref, w2_ref, b2_ref, out_ref, hbuf_ref):
    nsteps = pl.num_programs(0) - 1
    _, nr, nl = hbuf_ref.shape
    nf = nl // _CELLS_PER_ROW
    step = pl.program_id(0)
    P = _PAIR

    @pl.when(step < nsteps)
    def _matmul():
        for j in range(P):
            x = xw_ref[j]
            h = jnp.dot(x, bd_ref[...], preferred_element_type=jnp.float32)
            slot = (step % 2) * P + j
            hbuf_ref[slot] = jnp.maximum(h + b1_ref[...], 0.0)

    @pl.when(step > 0)
    def _search():
        prev = 1 - (step % 2)
        li = lax.broadcasted_iota(jnp.int32, (nl, nl), 0)
        mi = lax.broadcasted_iota(jnp.int32, (nl, nl), 1)
        foldm = jnp.where((li % nf) == (mi % nf), 1.0, 0.0).astype(jnp.float32)

        def count_ge(t_bits):
            t = lax.bitcast_convert_type(t_bits, jnp.float32)
            cnt = jnp.concatenate(
                [
                    jnp.sum(
                        (hbuf_ref[prev * P + j] >= t[j : j + 1]).astype(
                            jnp.float32
                        ),
                        axis=0,
                        keepdims=True,
                    )
                    for j in range(P)
                ],
                axis=0,
            )
            return jnp.dot(cnt, foldm, preferred_element_type=jnp.float32)

        def bs_body(_, carry):
            lo, hi = carry
            mid = lo + lax.div(hi - lo, 2)
            pred = count_ge(mid) >= float(_K_TOP)
            return jnp.where(pred, mid, lo), jnp.where(pred, hi, mid)

        lo0 = jnp.zeros((P, nl), jnp.int32)
        hi0 = jnp.full((P, nl), jnp.int32(2**31 - 1))
        lo, hi = lax.fori_loop(0, 31, bs_body, (lo0, hi0))

        t_lo = lax.bitcast_convert_type(lo, jnp.float32)
        t_hi = lax.bitcast_convert_type(hi, jnp.float32)
        sums_l, cgt_l = [], []
        for j in range(P):
            hh = hbuf_ref[prev * P + j]
            mgt = hh >= t_hi[j : j + 1]
            sums_l.append(
                jnp.sum(jnp.where(mgt, hh, 0.0), axis=0, keepdims=True)
            )
            cgt_l.append(
                jnp.sum(mgt.astype(jnp.float32), axis=0, keepdims=True)
            )
        sums = jnp.concatenate(sums_l, axis=0)
        cgt = jnp.concatenate(cgt_l, axis=0)
        sumsf = jnp.dot(sums, foldm, preferred_element_type=jnp.float32)
        cgtf = jnp.dot(cgt, foldm, preferred_element_type=jnp.float32)
        sum_top = sumsf + (float(_K_TOP) - cgtf) * t_lo
        pooled = sum_top[:, :nf] * (1.0 / _K_TOP)

        z = jnp.sum(pooled * w2_ref[...], axis=1, keepdims=True) + b2_ref[...]
        sig = (1.0 / (1.0 + jnp.exp(-z))).reshape(P, 1, 1)
        out_ref[pl.ds((step - 1) * P, P)] = sig


def _build_call(B, NR, D, F):
    C = _CELLS_PER_ROW
    P = _PAIR
    nsteps = B // P
    return pl.pallas_call(
        _cellcnn_body,
        grid=(nsteps + 1,),
        in_specs=[
            pl.BlockSpec(
                (P, NR, C * D), lambda b: (jnp.minimum(b, B // P - 1), 0, 0)
            ),
            pl.BlockSpec((C * D, C * F), lambda b: (0, 0)),
            pl.BlockSpec((1, C * F), lambda b: (0, 0)),
            pl.BlockSpec((1, F), lambda b: (0, 0)),
            pl.BlockSpec((1, 1), lambda b: (0, 0)),
        ],
        out_specs=pl.BlockSpec((B, 1, 1), lambda b: (0, 0, 0)),
        out_shape=jax.ShapeDtypeStruct((B, 1, 1), jnp.float32),
        scratch_shapes=[pltpu.VMEM((2 * P, NR, C * F), jnp.float32)],
    )


def kernel(inputs, W1, b1, W2, b2):
    B, N, D = inputs.shape
    F = W1.shape[1]
    C = _CELLS_PER_ROW
    NR = N // C
    xw = inputs.reshape(B, NR, C * D)
    eye = jnp.eye(C, dtype=W1.dtype)
    bd = jnp.einsum("ce,df->cdef", eye, W1).reshape(C * D, C * F)
    b1t = jnp.tile(b1, C).reshape(1, C * F)
    w2t = W2.reshape(1, F)
    b2r = b2.reshape(1, 1)
    out = _build_call(B, NR, D, F)(xw, bd, b1t, w2t, b2r)
    return out.reshape(B, 1)


# speedup vs baseline: 1.4463x; 1.1726x over previous
import jax
import jax.numpy as jnp
from jax import lax
from jax.experimental import pallas as pl
from jax.experimental.pallas import tpu as pltpu

_K_TOP = 256
_LANES = 128
_GROUP = 8
_NCHUNK = 4


def _cellcnn_body(G, x_ref, w1p_ref, b1p_ref, w2_ref, b2_ref, out_ref, hall_ref):
    nplane, ncell, nl = hall_ref.shape
    nf = nl // G
    nslab = x_ref.shape[1]
    step = pl.program_id(0)
    nsteps = nplane * G * _NCHUNK

    @pl.when(step < nsteps)
    def _matmul():
        b = step // _NCHUNK
        c = step % _NCHUNK
        p = b // G
        gpos = b % G
        x = x_ref[0]
        h = jnp.dot(x, w1p_ref[0], preferred_element_type=jnp.float32)
        h = jnp.maximum(h + b1p_ref[0], 0.0)
        sl = pl.ds(c * nslab, nslab)

        @pl.when(gpos == 0)
        def _init():
            hall_ref[p, sl, :] = h

        @pl.when(gpos > 0)
        def _accum():
            hall_ref[p, sl, :] = hall_ref[p, sl, :] + h

    @pl.when(step == nsteps)
    def _search():
        kf = float(_K_TOP)
        ncc = min(4096, ncell)

        def count_ge(t_bits):
            t = lax.bitcast_convert_type(t_bits, jnp.float32)
            rows = []
            for p in range(nplane):
                acc = jnp.zeros((1, nl), jnp.float32)
                for c in range(ncell // ncc):
                    blk = hall_ref[p, pl.ds(c * ncc, ncc), :]
                    acc = acc + jnp.sum(
                        (blk >= t[p : p + 1]).astype(jnp.float32),
                        axis=0,
                        keepdims=True,
                    )
                rows.append(acc)
            return jnp.concatenate(rows, axis=0)

        def bs_body(_, carry):
            lo, hi = carry
            mid = lo + lax.div(hi - lo, 2)
            pred = count_ge(mid) >= kf
            return jnp.where(pred, mid, lo), jnp.where(pred, hi, mid)

        lo0 = jnp.zeros((nplane, nl), jnp.int32)
        hi0 = jnp.full((nplane, nl), jnp.int32(2**31 - 1))
        lo, hi = lax.fori_loop(0, 31, bs_body, (lo0, hi0))

        t_lo = lax.bitcast_convert_type(lo, jnp.float32)
        t_hi = lax.bitcast_convert_type(hi, jnp.float32)
        sums_l, cgt_l = [], []
        for p in range(nplane):
            sacc = jnp.zeros((1, nl), jnp.float32)
            cacc = jnp.zeros((1, nl), jnp.float32)
            for c in range(ncell // ncc):
                blk = hall_ref[p, pl.ds(c * ncc, ncc), :]
                mgt = blk >= t_hi[p : p + 1]
                sacc = sacc + jnp.sum(
                    jnp.where(mgt, blk, 0.0), axis=0, keepdims=True
                )
                cacc = cacc + jnp.sum(
                    mgt.astype(jnp.float32), axis=0, keepdims=True
                )
            sums_l.append(sacc)
            cgt_l.append(cacc)
        sums = jnp.concatenate(sums_l, axis=0)
        cgt = jnp.concatenate(cgt_l, axis=0)
        sum_top = sums + (kf - cgt) * t_lo
        zraw = sum_top * (1.0 / kf) * w2_ref[...]

        li = lax.broadcasted_iota(jnp.int32, (nl, G), 0)
        gi = lax.broadcasted_iota(jnp.int32, (nl, G), 1)
        gmat = jnp.where((li // nf) == gi, 1.0, 0.0).astype(jnp.float32)
        z = jnp.dot(zraw, gmat, preferred_element_type=jnp.float32)
        z = z + b2_ref[...]
        out_ref[...] = 1.0 / (1.0 + jnp.exp(-z))


def _build_call(B, N, D, F, G, LANES):
    NC = _NCHUNK
    NP = B // G
    nslab = N // NC
    nsteps = B * NC

    def xmap(g):
        gc = jnp.minimum(g, nsteps - 1)
        return (gc // NC, gc % NC, 0)

    def wmap(g):
        gc = jnp.minimum(g, nsteps - 1)
        return ((gc // NC) % G, 0, 0)

    import functools
    return pl.pallas_call(
        functools.partial(_cellcnn_body, G),
        grid=(nsteps + 1,),
        in_specs=[
            pl.BlockSpec((1, nslab, D), xmap),
            pl.BlockSpec((1, D, LANES), wmap),
            pl.BlockSpec((1, 1, LANES), wmap),
            pl.BlockSpec((1, LANES), lambda g: (0, 0)),
            pl.BlockSpec((NP, G), lambda g: (0, 0)),
        ],
        out_specs=pl.BlockSpec((NP, G), lambda g: (0, 0)),
        out_shape=jax.ShapeDtypeStruct((NP, G), jnp.float32),
        scratch_shapes=[pltpu.VMEM((NP, N, LANES), jnp.float32)],
    )


def kernel(inputs, W1, b1, W2, b2):
    B, N, D = inputs.shape
    F = W1.shape[1]
    G = min(_GROUP, B)
    LANES = G * F
    eye = jnp.eye(G, dtype=W1.dtype)
    w1p = jnp.einsum("jk,df->jdkf", eye, W1).reshape(G, D, LANES)
    b1p = jnp.einsum("jk,f->jkf", eye, b1).reshape(G, 1, LANES)
    w2t = jnp.tile(W2[:, 0], G).reshape(1, LANES)
    b2r = jnp.broadcast_to(b2.reshape(1, 1), (B // G, G))
    out = _build_call(B, N, D, F, G, LANES)(inputs, w1p, b1p, w2t, b2r)
    return out.reshape(B, 1)
